# Initial kernel scaffold; baseline (speedup 1.0000x reference)
import jax
import jax.numpy as jnp
from jax.experimental import pallas as pl


def kernel(token_ids, emb0, emb1, emb2):
    raise NotImplementedError("write your pallas kernel here")


# speedup vs baseline: 4.1300x; 4.1300x over previous
import functools

import jax
import jax.numpy as jnp
from jax import lax
from jax.experimental import pallas as pl
from jax.experimental.pallas import tpu as pltpu
from jax.experimental.pallas import tpu_sc as plsc

BASE_VOCAB = 100000
EMBED_DIM = 64

_info = plsc.get_sparse_core_info()
_NC, _NS, _L = _info.num_cores, _info.num_subcores, _info.num_lanes
_NW = _NC * _NS

_CHUNK = 1024
_SUB = 128
_NSUB = _CHUNK // _SUB


def _gather_kernel(n_total, idx_hbm, table_hbm, out_hbm, idx_v, rows_v, gsem):
    b_per_w = n_total // _NW
    n_chunks = b_per_w // _CHUNK
    wid = lax.axis_index("s") * _NC + lax.axis_index("c")
    base = wid * b_per_w

    vmax = jnp.full((_L,), BASE_VOCAB - 1, dtype=jnp.int32)
    vmin = jnp.zeros((_L,), dtype=jnp.int32)

    def body(i, _):
        start = base + i * _CHUNK
        row = pl.multiple_of(start // _SUB, 8)
        pltpu.sync_copy(idx_hbm.at[pl.ds(row, _NSUB)], idx_v)
        for j in range(_NSUB):
            for c in range(_SUB // _L):
                sl = (j, pl.ds(c * _L, _L))
                idx_v[sl] = jnp.minimum(jnp.maximum(idx_v[sl], vmin), vmax)
        copies = []
        for j in range(_NSUB):
            copies.append(pltpu.async_copy(
                table_hbm.at[idx_v.at[j]],
                rows_v.at[pl.ds(j * _SUB, _SUB)],
                gsem))
        for cp in copies:
            cp.wait()
        pltpu.sync_copy(rows_v, out_hbm.at[pl.ds(start, _CHUNK)])
        return ()

    lax.fori_loop(0, n_chunks, body, ())


@functools.partial(jax.jit, static_argnames=("n_total",))
def _embedding_gather(flat_ids, table, n_total):
    mesh = plsc.VectorSubcoreMesh(core_axis_name="c", subcore_axis_name="s")
    kern = functools.partial(
        pl.kernel,
        out_type=jax.ShapeDtypeStruct((n_total, EMBED_DIM), jnp.float32),
        mesh=mesh,
        compiler_params=pltpu.CompilerParams(use_tc_tiling_on_sc=False),
        scratch_types=[
            pltpu.VMEM((_NSUB, _SUB), jnp.int32),
            pltpu.VMEM((_CHUNK, EMBED_DIM), jnp.float32),
            pltpu.SemaphoreType.DMA,
        ],
    )(functools.partial(_gather_kernel, n_total))
    return kern(flat_ids, table)


def kernel(token_ids, emb0, emb1, emb2):
    B, T = token_ids.shape
    n_total = B * T
    flat_ids = token_ids.reshape(n_total // _SUB, _SUB).astype(jnp.int32)
    out = _embedding_gather(flat_ids, emb0, n_total)
    return out.reshape(B, T, EMBED_DIM)


# speedup vs baseline: 4.1795x; 1.0120x over previous
import functools

import jax
import jax.numpy as jnp
from jax import lax
from jax.experimental import pallas as pl
from jax.experimental.pallas import tpu as pltpu
from jax.experimental.pallas import tpu_sc as plsc

BASE_VOCAB = 100000
EMBED_DIM = 64

_info = plsc.get_sparse_core_info()
_NC, _NS, _L = _info.num_cores, _info.num_subcores, _info.num_lanes
_NW = _NC * _NS

_CHUNK = 512
_SUB = 128
_NSUB = _CHUNK // _SUB


def _gather_kernel(n_total, idx_hbm, table_hbm, out_hbm, idx0, idx1, rows0,
                   rows1, gsem):
    b_per_w = n_total // _NW
    n_chunks = b_per_w // _CHUNK
    n_pairs = n_chunks // 2
    wid = lax.axis_index("s") * _NC + lax.axis_index("c")
    base = wid * b_per_w

    vmax = jnp.full((_L,), BASE_VOCAB - 1, dtype=jnp.int32)
    vmin = jnp.zeros((_L,), dtype=jnp.int32)

    idx_bufs = (idx0, idx1)
    rows_bufs = (rows0, rows1)

    def load_clamp_fire(i, idx_v, rows_v):
        start = base + i * _CHUNK
        pltpu.sync_copy(idx_hbm.at[pl.ds(start // _SUB, _NSUB)], idx_v)
        for j in range(_NSUB):
            for c in range(_SUB // _L):
                sl = (j, pl.ds(c * _L, _L))
                idx_v[sl] = jnp.minimum(jnp.maximum(idx_v[sl], vmin), vmax)
        for j in range(_NSUB):
            pltpu.async_copy(
                table_hbm.at[idx_v.at[j]],
                rows_v.at[pl.ds(j * _SUB, _SUB)],
                gsem)

    def wait_gathers(idx_v, rows_v):
        for j in range(_NSUB):
            pltpu.make_async_copy(
                table_hbm.at[idx_v.at[j]],
                rows_v.at[pl.ds(j * _SUB, _SUB)],
                gsem).wait()

    load_clamp_fire(0, idx0, rows0)

    def body(p, _):
        for b in range(2):
            i = 2 * p + b
            idx_v, rows_v = idx_bufs[b], rows_bufs[b]
            idx_n, rows_n = idx_bufs[1 - b], rows_bufs[1 - b]

            @pl.when(i < n_chunks - 1)
            def _():
                load_clamp_fire(i + 1, idx_n, rows_n)

            wait_gathers(idx_v, rows_v)
            pltpu.sync_copy(rows_v, out_hbm.at[pl.ds(base + i * _CHUNK, _CHUNK)])
        return ()

    lax.fori_loop(0, n_pairs, body, ())


@functools.partial(jax.jit, static_argnames=("n_total",))
def _embedding_gather(flat_ids, table, n_total):
    mesh = plsc.VectorSubcoreMesh(core_axis_name="c", subcore_axis_name="s")
    kern = functools.partial(
        pl.kernel,
        out_type=jax.ShapeDtypeStruct((n_total, EMBED_DIM), jnp.float32),
        mesh=mesh,
        compiler_params=pltpu.CompilerParams(use_tc_tiling_on_sc=False),
        scratch_types=[
            pltpu.VMEM((_NSUB, _SUB), jnp.int32),
            pltpu.VMEM((_NSUB, _SUB), jnp.int32),
            pltpu.VMEM((_CHUNK, EMBED_DIM), jnp.float32),
            pltpu.VMEM((_CHUNK, EMBED_DIM), jnp.float32),
            pltpu.SemaphoreType.DMA,
        ],
    )(functools.partial(_gather_kernel, n_total))
    return kern(flat_ids, table)


def kernel(token_ids, emb0, emb1, emb2):
    B, T = token_ids.shape
    n_total = B * T
    flat_ids = token_ids.reshape(n_total // _SUB, _SUB).astype(jnp.int32)
    out = _embedding_gather(flat_ids, emb0, n_total)
    return out.reshape(B, T, EMBED_DIM)


# speedup vs baseline: 4.2063x; 1.0064x over previous
import functools

import jax
import jax.numpy as jnp
from jax import lax
from jax.experimental import pallas as pl
from jax.experimental.pallas import tpu as pltpu
from jax.experimental.pallas import tpu_sc as plsc

BASE_VOCAB = 100000
EMBED_DIM = 64
SEQ = 200

_info = plsc.get_sparse_core_info()
_NC, _NS, _L = _info.num_cores, _info.num_subcores, _info.num_lanes
_NW = _NC * _NS

_NB = 4
_CHUNK = _NB * SEQ
_SPLITS = ((0, 128), (128, 72))


def _gather_kernel(batch, idx_hbm, table_hbm, out_hbm, idx0, idx1, rows0,
                   rows1, gsem):
    b_per_w = batch // _NW
    n_chunks = b_per_w // _NB
    n_pairs = n_chunks // 2
    wid = lax.axis_index("s") * _NC + lax.axis_index("c")
    b_base = wid * b_per_w

    vmax = jnp.full((_L,), BASE_VOCAB - 1, dtype=jnp.int32)
    vmin = jnp.zeros((_L,), dtype=jnp.int32)

    idx_bufs = (idx0, idx1)
    rows_bufs = (rows0, rows1)

    def load_clamp_fire(i, idx_v, rows_v):
        b0 = b_base + i * _NB
        pltpu.sync_copy(idx_hbm.at[pl.ds(b0 * SEQ, _CHUNK)], idx_v)
        for c in range(_CHUNK // _L):
            sl = pl.ds(c * _L, _L)
            idx_v[sl] = jnp.minimum(jnp.maximum(idx_v[sl], vmin), vmax)
        for b in range(_NB):
            for (t0, n) in _SPLITS:
                pltpu.async_copy(
                    table_hbm.at[idx_v.at[pl.ds(b * SEQ + t0, n)]],
                    rows_v.at[b].at[pl.ds(t0, n)],
                    gsem)

    def wait_gathers(idx_v, rows_v):
        for b in range(_NB):
            for (t0, n) in _SPLITS:
                pltpu.make_async_copy(
                    table_hbm.at[idx_v.at[pl.ds(b * SEQ + t0, n)]],
                    rows_v.at[b].at[pl.ds(t0, n)],
                    gsem).wait()

    load_clamp_fire(0, idx0, rows0)

    def body(p, _):
        for par in range(2):
            i = 2 * p + par
            idx_v, rows_v = idx_bufs[par], rows_bufs[par]
            idx_n, rows_n = idx_bufs[1 - par], rows_bufs[1 - par]

            @pl.when(i < n_chunks - 1)
            def _():
                load_clamp_fire(i + 1, idx_n, rows_n)

            wait_gathers(idx_v, rows_v)
            pltpu.sync_copy(rows_v, out_hbm.at[pl.ds(b_base + i * _NB, _NB)])
        return ()

    lax.fori_loop(0, n_pairs, body, ())


@functools.partial(jax.jit, static_argnames=("batch",))
def _embedding_gather(flat_ids, table, batch):
    mesh = plsc.VectorSubcoreMesh(core_axis_name="c", subcore_axis_name="s")
    kern = functools.partial(
        pl.kernel,
        out_type=jax.ShapeDtypeStruct((batch, SEQ, EMBED_DIM), jnp.float32),
        mesh=mesh,
        compiler_params=pltpu.CompilerParams(use_tc_tiling_on_sc=False),
        scratch_types=[
            pltpu.VMEM((_CHUNK,), jnp.int32),
            pltpu.VMEM((_CHUNK,), jnp.int32),
            pltpu.VMEM((_NB, SEQ, EMBED_DIM), jnp.float32),
            pltpu.VMEM((_NB, SEQ, EMBED_DIM), jnp.float32),
            pltpu.SemaphoreType.DMA,
        ],
    )(functools.partial(_gather_kernel, batch))
    return kern(flat_ids, table)


def kernel(token_ids, emb0, emb1, emb2):
    B, T = token_ids.shape
    flat_ids = token_ids.reshape(B * T).astype(jnp.int32)
    return _embedding_gather(flat_ids, emb0, B)


# speedup vs baseline: 7.1791x; 1.7067x over previous
import functools

import jax
import jax.numpy as jnp
from jax import lax
from jax.experimental import pallas as pl
from jax.experimental.pallas import tpu as pltpu
from jax.experimental.pallas import tpu_sc as plsc

BASE_VOCAB = 100000
EMBED_DIM = 64
PADDED_DIM = 128
SEQ = 200

_info = plsc.get_sparse_core_info()
_NC, _NS, _L = _info.num_cores, _info.num_subcores, _info.num_lanes
_NW = _NC * _NS

_NB = 2
_CHUNK = _NB * SEQ
_SPLITS = ((0, 128), (128, 72))


def _gather_kernel(batch, idx_hbm, table_hbm, out_hbm, idx0, idx1, rows0,
                   rows1, gsem):
    b_per_w = batch // _NW
    n_chunks = b_per_w // _NB
    n_pairs = n_chunks // 2
    wid = lax.axis_index("s") * _NC + lax.axis_index("c")
    b_base = wid * b_per_w

    vmax = jnp.full((_L,), BASE_VOCAB - 1, dtype=jnp.int32)
    vmin = jnp.zeros((_L,), dtype=jnp.int32)

    idx_bufs = (idx0, idx1)
    rows_bufs = (rows0, rows1)

    def load_clamp_fire(i, idx_v, rows_v):
        b0 = b_base + i * _NB
        pltpu.sync_copy(idx_hbm.at[pl.ds(b0 * SEQ, _CHUNK)], idx_v)
        for c in range(_CHUNK // _L):
            sl = pl.ds(c * _L, _L)
            idx_v[sl] = jnp.minimum(jnp.maximum(idx_v[sl], vmin), vmax)
        for b in range(_NB):
            for (t0, n) in _SPLITS:
                pltpu.async_copy(
                    table_hbm.at[idx_v.at[pl.ds(b * SEQ + t0, n)]],
                    rows_v.at[b].at[pl.ds(t0, n)],
                    gsem)

    def wait_gathers(idx_v, rows_v):
        for b in range(_NB):
            for (t0, n) in _SPLITS:
                pltpu.make_async_copy(
                    table_hbm.at[idx_v.at[pl.ds(b * SEQ + t0, n)]],
                    rows_v.at[b].at[pl.ds(t0, n)],
                    gsem).wait()

    load_clamp_fire(0, idx0, rows0)

    def body(p, _):
        for par in range(2):
            i = 2 * p + par
            idx_v, rows_v = idx_bufs[par], rows_bufs[par]
            idx_n, rows_n = idx_bufs[1 - par], rows_bufs[1 - par]

            @pl.when(i < n_chunks - 1)
            def _():
                load_clamp_fire(i + 1, idx_n, rows_n)

            wait_gathers(idx_v, rows_v)
            for b in range(_NB):
                pltpu.sync_copy(
                    rows_v.at[b],
                    out_hbm.at[b_base + i * _NB + b].at[:, pl.ds(0, EMBED_DIM)])
        return ()

    lax.fori_loop(0, n_pairs, body, ())


@functools.partial(jax.jit, static_argnames=("batch",))
def _embedding_gather(flat_ids, table, batch):
    mesh = plsc.VectorSubcoreMesh(core_axis_name="c", subcore_axis_name="s")
    kern = functools.partial(
        pl.kernel,
        out_type=jax.ShapeDtypeStruct((batch, SEQ, PADDED_DIM), jnp.float32),
        mesh=mesh,
        compiler_params=pltpu.CompilerParams(use_tc_tiling_on_sc=False),
        scratch_types=[
            pltpu.VMEM((_CHUNK,), jnp.int32),
            pltpu.VMEM((_CHUNK,), jnp.int32),
            pltpu.VMEM((_NB, SEQ, EMBED_DIM), jnp.float32),
            pltpu.VMEM((_NB, SEQ, EMBED_DIM), jnp.float32),
            pltpu.SemaphoreType.DMA,
        ],
    )(functools.partial(_gather_kernel, batch))
    return kern(flat_ids, table)


def kernel(token_ids, emb0, emb1, emb2):
    B, T = token_ids.shape
    flat_ids = token_ids.reshape(B * T).astype(jnp.int32)
    out_padded = _embedding_gather(flat_ids, emb0, B)
    return out_padded[:, :, :EMBED_DIM]


# speedup vs baseline: 7.3569x; 1.0248x over previous
import functools

import jax
import jax.numpy as jnp
from jax import lax
from jax.experimental import pallas as pl
from jax.experimental.pallas import tpu as pltpu
from jax.experimental.pallas import tpu_sc as plsc

BASE_VOCAB = 100000
EMBED_DIM = 64
PADDED_DIM = 128
SEQ = 200

_info = plsc.get_sparse_core_info()
_NC, _NS, _L = _info.num_cores, _info.num_subcores, _info.num_lanes
_NW = _NC * _NS

_NB = 4
_CHUNK = _NB * SEQ
_SPLITS = ((0, 128), (128, 72))


def _gather_kernel(batch, idx_hbm, table_hbm, out_hbm, idx0, idx1, rows0,
                   rows1, gsem):
    b_per_w = batch // _NW
    n_chunks = b_per_w // _NB
    n_pairs = n_chunks // 2
    wid = lax.axis_index("s") * _NC + lax.axis_index("c")
    b_base = wid * b_per_w

    vmax = jnp.full((_L,), BASE_VOCAB - 1, dtype=jnp.int32)
    vmin = jnp.zeros((_L,), dtype=jnp.int32)

    idx_bufs = (idx0, idx1)
    rows_bufs = (rows0, rows1)

    def load_clamp_fire(i, idx_v, rows_v):
        b0 = b_base + i * _NB
        pltpu.sync_copy(idx_hbm.at[pl.ds(b0 * SEQ, _CHUNK)], idx_v)
        for c in range(_CHUNK // _L):
            sl = pl.ds(c * _L, _L)
            idx_v[sl] = jnp.minimum(jnp.maximum(idx_v[sl], vmin), vmax)
        for b in range(_NB):
            for (t0, n) in _SPLITS:
                pltpu.async_copy(
                    table_hbm.at[idx_v.at[pl.ds(b * SEQ + t0, n)]],
                    rows_v.at[b].at[pl.ds(t0, n)],
                    gsem)

    def wait_gathers(idx_v, rows_v):
        for b in range(_NB):
            for (t0, n) in _SPLITS:
                pltpu.make_async_copy(
                    table_hbm.at[idx_v.at[pl.ds(b * SEQ + t0, n)]],
                    rows_v.at[b].at[pl.ds(t0, n)],
                    gsem).wait()

    load_clamp_fire(0, idx0, rows0)

    def body(p, _):
        for par in range(2):
            i = 2 * p + par
            idx_v, rows_v = idx_bufs[par], rows_bufs[par]
            idx_n, rows_n = idx_bufs[1 - par], rows_bufs[1 - par]

            @pl.when(i < n_chunks - 1)
            def _():
                load_clamp_fire(i + 1, idx_n, rows_n)

            wait_gathers(idx_v, rows_v)
            for b in range(_NB):
                pltpu.sync_copy(
                    rows_v.at[b],
                    out_hbm.at[b_base + i * _NB + b].at[:, pl.ds(0, EMBED_DIM)])
        return ()

    lax.fori_loop(0, n_pairs, body, ())


@functools.partial(jax.jit, static_argnames=("batch",))
def _embedding_gather(flat_ids, table, batch):
    mesh = plsc.VectorSubcoreMesh(core_axis_name="c", subcore_axis_name="s")
    kern = functools.partial(
        pl.kernel,
        out_type=jax.ShapeDtypeStruct((batch, SEQ, PADDED_DIM), jnp.float32),
        mesh=mesh,
        compiler_params=pltpu.CompilerParams(use_tc_tiling_on_sc=False),
        scratch_types=[
            pltpu.VMEM((_CHUNK,), jnp.int32),
            pltpu.VMEM((_CHUNK,), jnp.int32),
            pltpu.VMEM((_NB, SEQ, EMBED_DIM), jnp.float32),
            pltpu.VMEM((_NB, SEQ, EMBED_DIM), jnp.float32),
            pltpu.SemaphoreType.DMA,
        ],
    )(functools.partial(_gather_kernel, batch))
    return kern(flat_ids, table)


def kernel(token_ids, emb0, emb1, emb2):
    B, T = token_ids.shape
    flat_ids = token_ids.reshape(B * T).astype(jnp.int32)
    out_padded = _embedding_gather(flat_ids, emb0, B)
    return out_padded[:, :, :EMBED_DIM]


# speedup vs baseline: 7.3631x; 1.0009x over previous
import functools

import jax
import jax.numpy as jnp
from jax import lax
from jax.experimental import pallas as pl
from jax.experimental.pallas import tpu as pltpu
from jax.experimental.pallas import tpu_sc as plsc

BASE_VOCAB = 100000
EMBED_DIM = 64
PADDED_DIM = 128
SEQ = 200

_info = plsc.get_sparse_core_info()
_NC, _NS, _L = _info.num_cores, _info.num_subcores, _info.num_lanes
_NW = _NC * _NS

_NB = 4
_CHUNK = _NB * SEQ
_SPLITS = ((0, 104), (104, 96))


def _gather_kernel(batch, idx_hbm, table_hbm, out_hbm, idx0, idx1, rows0,
                   rows1, gsem):
    b_per_w = batch // _NW
    n_chunks = b_per_w // _NB
    n_pairs = n_chunks // 2
    wid = lax.axis_index("s") * _NC + lax.axis_index("c")
    b_base = wid * b_per_w

    vmax = jnp.full((_L,), BASE_VOCAB - 1, dtype=jnp.int32)
    vmin = jnp.zeros((_L,), dtype=jnp.int32)

    idx_bufs = (idx0, idx1)
    rows_bufs = (rows0, rows1)

    def load_clamp_fire(i, idx_v, rows_v):
        b0 = b_base + i * _NB
        pltpu.sync_copy(idx_hbm.at[pl.ds(b0 * SEQ, _CHUNK)], idx_v)
        for c in range(_CHUNK // _L):
            sl = pl.ds(c * _L, _L)
            idx_v[sl] = jnp.minimum(jnp.maximum(idx_v[sl], vmin), vmax)
        for b in range(_NB):
            for (t0, n) in _SPLITS:
                pltpu.async_copy(
                    table_hbm.at[idx_v.at[pl.ds(b * SEQ + t0, n)]],
                    rows_v.at[b].at[pl.ds(t0, n)],
                    gsem)

    def wait_gathers(idx_v, rows_v):
        for b in range(_NB):
            for (t0, n) in _SPLITS:
                pltpu.make_async_copy(
                    table_hbm.at[idx_v.at[pl.ds(b * SEQ + t0, n)]],
                    rows_v.at[b].at[pl.ds(t0, n)],
                    gsem).wait()

    load_clamp_fire(0, idx0, rows0)

    def body(p, _):
        for par in range(2):
            i = 2 * p + par
            idx_v, rows_v = idx_bufs[par], rows_bufs[par]
            idx_n, rows_n = idx_bufs[1 - par], rows_bufs[1 - par]

            @pl.when(i < n_chunks - 1)
            def _():
                load_clamp_fire(i + 1, idx_n, rows_n)

            wait_gathers(idx_v, rows_v)
            for b in range(_NB):
                pltpu.sync_copy(
                    rows_v.at[b],
                    out_hbm.at[b_base + i * _NB + b].at[:, pl.ds(0, EMBED_DIM)])
        return ()

    lax.fori_loop(0, n_pairs, body, ())


@functools.partial(jax.jit, static_argnames=("batch",))
def _embedding_gather(flat_ids, table, batch):
    mesh = plsc.VectorSubcoreMesh(core_axis_name="c", subcore_axis_name="s")
    kern = functools.partial(
        pl.kernel,
        out_type=jax.ShapeDtypeStruct((batch, SEQ, PADDED_DIM), jnp.float32),
        mesh=mesh,
        compiler_params=pltpu.CompilerParams(use_tc_tiling_on_sc=False),
        scratch_types=[
            pltpu.VMEM((_CHUNK,), jnp.int32),
            pltpu.VMEM((_CHUNK,), jnp.int32),
            pltpu.VMEM((_NB, SEQ, EMBED_DIM), jnp.float32),
            pltpu.VMEM((_NB, SEQ, EMBED_DIM), jnp.float32),
            pltpu.SemaphoreType.DMA,
        ],
    )(functools.partial(_gather_kernel, batch))
    return kern(flat_ids, table)


def kernel(token_ids, emb0, emb1, emb2):
    B, T = token_ids.shape
    flat_ids = token_ids.reshape(B * T).astype(jnp.int32)
    out_padded = _embedding_gather(flat_ids, emb0, B)
    return out_padded[:, :, :EMBED_DIM]


# speedup vs baseline: 7.5255x; 1.0221x over previous
import functools

import jax
import jax.numpy as jnp
from jax import lax
from jax.experimental import pallas as pl
from jax.experimental.pallas import tpu as pltpu
from jax.experimental.pallas import tpu_sc as plsc

BASE_VOCAB = 100000
EMBED_DIM = 64
PADDED_DIM = 128
SEQ = 200

_info = plsc.get_sparse_core_info()
_NC, _NS, _L = _info.num_cores, _info.num_subcores, _info.num_lanes
_NW = _NC * _NS

_NB = 4
_CHUNK = _NB * SEQ
_SPLITS = ((0, 104), (104, 96))


def _gather_kernel(batch, idx_hbm, table_hbm, out_hbm, idx0, idx1, rows0,
                   rows1, isem, gsem, wsem):
    b_per_w = batch // _NW
    n_chunks = b_per_w // _NB
    n_pairs = n_chunks // 2
    wid = lax.axis_index("s") * _NC + lax.axis_index("c")
    b_base = wid * b_per_w

    vmax = jnp.full((_L,), BASE_VOCAB - 1, dtype=jnp.int32)
    vmin = jnp.zeros((_L,), dtype=jnp.int32)

    idx_bufs = (idx0, idx1)
    rows_bufs = (rows0, rows1)

    def idx_load(i, idx_v):
        return pltpu.async_copy(
            idx_hbm.at[pl.ds((b_base + i * _NB) * SEQ, _CHUNK)], idx_v, isem)

    def clamp_fire(i, idx_v, rows_v):
        for c in range(_CHUNK // _L):
            sl = pl.ds(c * _L, _L)
            idx_v[sl] = jnp.minimum(jnp.maximum(idx_v[sl], vmin), vmax)
        for b in range(_NB):
            for (t0, n) in _SPLITS:
                pltpu.async_copy(
                    table_hbm.at[idx_v.at[pl.ds(b * SEQ + t0, n)]],
                    rows_v.at[b].at[pl.ds(t0, n)],
                    gsem)

    def wait_gathers(idx_v, rows_v):
        for b in range(_NB):
            for (t0, n) in _SPLITS:
                pltpu.make_async_copy(
                    table_hbm.at[idx_v.at[pl.ds(b * SEQ + t0, n)]],
                    rows_v.at[b].at[pl.ds(t0, n)],
                    gsem).wait()

    def write_out(i, rows_v):
        for b in range(_NB):
            pltpu.async_copy(
                rows_v.at[b],
                out_hbm.at[b_base + i * _NB + b].at[:, pl.ds(0, EMBED_DIM)],
                wsem)

    def drain_write(i, rows_v):
        for b in range(_NB):
            pltpu.make_async_copy(
                rows_v.at[b],
                out_hbm.at[b_base + i * _NB + b].at[:, pl.ds(0, EMBED_DIM)],
                wsem).wait()

    idx_load(0, idx0).wait()
    clamp_fire(0, idx0, rows0)
    idx_load(1, idx1)

    def body(p, _):
        for par in range(2):
            i = 2 * p + par
            idx_v, rows_v = idx_bufs[par], rows_bufs[par]
            idx_n, rows_n = idx_bufs[1 - par], rows_bufs[1 - par]

            @pl.when(i >= 1)
            def _():
                drain_write(i - 1, rows_n)

            @pl.when(i < n_chunks - 1)
            def _():
                pltpu.make_async_copy(
                    idx_hbm.at[pl.ds((b_base + (i + 1) * _NB) * SEQ, _CHUNK)],
                    idx_n, isem).wait()
                clamp_fire(i + 1, idx_n, rows_n)

            wait_gathers(idx_v, rows_v)

            @pl.when(i < n_chunks - 2)
            def _():
                idx_load(i + 2, idx_v)

            write_out(i, rows_v)
        return ()

    lax.fori_loop(0, n_pairs, body, ())
    drain_write(n_chunks - 1, rows_bufs[(n_chunks - 1) % 2])


@functools.partial(jax.jit, static_argnames=("batch",))
def _embedding_gather(flat_ids, table, batch):
    mesh = plsc.VectorSubcoreMesh(core_axis_name="c", subcore_axis_name="s")
    kern = functools.partial(
        pl.kernel,
        out_type=jax.ShapeDtypeStruct((batch, SEQ, PADDED_DIM), jnp.float32),
        mesh=mesh,
        compiler_params=pltpu.CompilerParams(use_tc_tiling_on_sc=False),
        scratch_types=[
            pltpu.VMEM((_CHUNK,), jnp.int32),
            pltpu.VMEM((_CHUNK,), jnp.int32),
            pltpu.VMEM((_NB, SEQ, EMBED_DIM), jnp.float32),
            pltpu.VMEM((_NB, SEQ, EMBED_DIM), jnp.float32),
            pltpu.SemaphoreType.DMA,
            pltpu.SemaphoreType.DMA,
            pltpu.SemaphoreType.DMA,
        ],
    )(functools.partial(_gather_kernel, batch))
    return kern(flat_ids, table)


def kernel(token_ids, emb0, emb1, emb2):
    B, T = token_ids.shape
    flat_ids = token_ids.reshape(B * T).astype(jnp.int32)
    out_padded = _embedding_gather(flat_ids, emb0, B)
    return out_padded[:, :, :EMBED_DIM]


# speedup vs baseline: 7.5336x; 1.0011x over previous
import functools

import jax
import jax.numpy as jnp
from jax import lax
from jax.experimental import pallas as pl
from jax.experimental.pallas import tpu as pltpu
from jax.experimental.pallas import tpu_sc as plsc

BASE_VOCAB = 100000
EMBED_DIM = 64
PADDED_DIM = 128
SEQ = 200

_info = plsc.get_sparse_core_info()
_NC, _NS, _L = _info.num_cores, _info.num_subcores, _info.num_lanes
_NW = _NC * _NS

_NB = 4
_CHUNK = _NB * SEQ
_SPLITS = ((0, 104), (104, 96))


def _gather_kernel(batch, idx_hbm, table_hbm, out_hbm, idx0, idx1, rows0,
                   rows1, isem, gsem, wsem):
    b_per_w = batch // _NW
    n_chunks = b_per_w // _NB
    n_pairs = n_chunks // 2
    wid = lax.axis_index("s") * _NC + lax.axis_index("c")
    b_base = wid * b_per_w

    vmax = jnp.full((_L,), BASE_VOCAB - 1, dtype=jnp.int32)
    vmin = jnp.zeros((_L,), dtype=jnp.int32)

    idx_bufs = (idx0, idx1)
    rows_bufs = (rows0, rows1)

    def idx_load(i, idx_v):
        return pltpu.async_copy(
            idx_hbm.at[pl.ds((b_base + i * _NB) * SEQ, _CHUNK)], idx_v, isem)

    def clamp_fire(i, idx_v, rows_v):
        for c in range(_CHUNK // _L):
            sl = pl.ds(c * _L, _L)
            idx_v[sl] = jnp.minimum(jnp.maximum(idx_v[sl], vmin), vmax)
        for b in range(_NB):
            for (t0, n) in _SPLITS:
                pltpu.async_copy(
                    table_hbm.at[idx_v.at[pl.ds(b * SEQ + t0, n)]],
                    rows_v.at[b].at[pl.ds(t0, n)],
                    gsem)

    def out_slice(i):
        return out_hbm.at[pl.ds(b_base + i * _NB, _NB)].at[:, :, pl.ds(0, EMBED_DIM)]

    def wait_gathers(i, rows_v):
        pltpu.make_async_copy(out_slice(i), rows_v, gsem).wait()

    def write_out(i, rows_v):
        pltpu.async_copy(rows_v, out_slice(i), wsem)

    def drain_write(i, rows_v):
        pltpu.make_async_copy(rows_v, out_slice(i), wsem).wait()

    idx_load(0, idx0).wait()
    clamp_fire(0, idx0, rows0)
    idx_load(1, idx1)

    def body(p, _):
        for par in range(2):
            i = 2 * p + par
            idx_v, rows_v = idx_bufs[par], rows_bufs[par]
            idx_n, rows_n = idx_bufs[1 - par], rows_bufs[1 - par]

            @pl.when(i >= 1)
            def _():
                drain_write(i - 1, rows_n)

            @pl.when(i < n_chunks - 1)
            def _():
                pltpu.make_async_copy(
                    idx_hbm.at[pl.ds((b_base + (i + 1) * _NB) * SEQ, _CHUNK)],
                    idx_n, isem).wait()
                clamp_fire(i + 1, idx_n, rows_n)

            wait_gathers(i, rows_v)

            @pl.when(i < n_chunks - 2)
            def _():
                idx_load(i + 2, idx_v)

            write_out(i, rows_v)
        return ()

    lax.fori_loop(0, n_pairs, body, ())
    drain_write(n_chunks - 1, rows_bufs[(n_chunks - 1) % 2])


@functools.partial(jax.jit, static_argnames=("batch",))
def _embedding_gather(flat_ids, table, batch):
    mesh = plsc.VectorSubcoreMesh(core_axis_name="c", subcore_axis_name="s")
    kern = functools.partial(
        pl.kernel,
        out_type=jax.ShapeDtypeStruct((batch, SEQ, PADDED_DIM), jnp.float32),
        mesh=mesh,
        compiler_params=pltpu.CompilerParams(use_tc_tiling_on_sc=False),
        scratch_types=[
            pltpu.VMEM((_CHUNK,), jnp.int32),
            pltpu.VMEM((_CHUNK,), jnp.int32),
            pltpu.VMEM((_NB, SEQ, EMBED_DIM), jnp.float32),
            pltpu.VMEM((_NB, SEQ, EMBED_DIM), jnp.float32),
            pltpu.SemaphoreType.DMA,
            pltpu.SemaphoreType.DMA,
            pltpu.SemaphoreType.DMA,
        ],
    )(functools.partial(_gather_kernel, batch))
    return kern(flat_ids, table)


def kernel(token_ids, emb0, emb1, emb2):
    B, T = token_ids.shape
    flat_ids = token_ids.reshape(B * T).astype(jnp.int32)
    out_padded = _embedding_gather(flat_ids, emb0, B)
    return out_padded[:, :, :EMBED_DIM]


# speedup vs baseline: 7.5447x; 1.0015x over previous
import functools

import jax
import jax.numpy as jnp
from jax import lax
from jax.experimental import pallas as pl
from jax.experimental.pallas import tpu as pltpu
from jax.experimental.pallas import tpu_sc as plsc

BASE_VOCAB = 100000
EMBED_DIM = 64
PADDED_DIM = 128
SEQ = 200

_info = plsc.get_sparse_core_info()
_NC, _NS, _L = _info.num_cores, _info.num_subcores, _info.num_lanes
_NW = _NC * _NS

_NB = 2
_CHUNK = _NB * SEQ
_SPLITS = ((0, 104), (104, 96))


def _gather_kernel(batch, idx_hbm, table_hbm, out_hbm, idx0, idx1, idx2,
                   rows0, rows1, rows2, isem, gsem, wsem):
    b_per_w = batch // _NW
    n_chunks = b_per_w // _NB
    wid = lax.axis_index("s") * _NC + lax.axis_index("c")
    b_base = wid * b_per_w

    vmax = jnp.full((_L,), BASE_VOCAB - 1, dtype=jnp.int32)
    vmin = jnp.zeros((_L,), dtype=jnp.int32)

    idx_bufs = (idx0, idx1, idx2)
    rows_bufs = (rows0, rows1, rows2)

    def idx_load(i, idx_v):
        return pltpu.async_copy(
            idx_hbm.at[pl.ds((b_base + i * _NB) * SEQ, _CHUNK)], idx_v, isem)

    def drain_idx(i, idx_v):
        pltpu.make_async_copy(
            idx_hbm.at[pl.ds((b_base + i * _NB) * SEQ, _CHUNK)], idx_v,
            isem).wait()

    def clamp_fire(i, idx_v, rows_v):
        for c in range(_CHUNK // _L):
            sl = pl.ds(c * _L, _L)
            idx_v[sl] = jnp.minimum(jnp.maximum(idx_v[sl], vmin), vmax)
        for b in range(_NB):
            for (t0, n) in _SPLITS:
                pltpu.async_copy(
                    table_hbm.at[idx_v.at[pl.ds(b * SEQ + t0, n)]],
                    rows_v.at[b].at[pl.ds(t0, n)],
                    gsem)

    def out_slice(i):
        return out_hbm.at[pl.ds(b_base + i * _NB, _NB)].at[:, :, pl.ds(0, EMBED_DIM)]

    def wait_gathers(i, rows_v):
        pltpu.make_async_copy(out_slice(i), rows_v, gsem).wait()

    def write_out(i, rows_v):
        pltpu.async_copy(rows_v, out_slice(i), wsem)

    def drain_write(i, rows_v):
        pltpu.make_async_copy(rows_v, out_slice(i), wsem).wait()

    idx_load(0, idx0).wait()
    clamp_fire(0, idx0, rows0)
    idx_load(1, idx1)

    def step(i, q):
        @pl.when(i >= 2)
        def _():
            drain_write(i - 2, rows_bufs[(q + 1) % 3])

        @pl.when(i < n_chunks - 1)
        def _():
            drain_idx(i + 1, idx_bufs[(q + 1) % 3])
            clamp_fire(i + 1, idx_bufs[(q + 1) % 3], rows_bufs[(q + 1) % 3])

        wait_gathers(i, rows_bufs[q])

        @pl.when(i < n_chunks - 2)
        def _():
            idx_load(i + 2, idx_bufs[(q + 2) % 3])

        write_out(i, rows_bufs[q])

    n_trips = (n_chunks - 1) // 3

    def body(p, _):
        for q in range(3):
            step(3 * p + q, q)
        return ()

    lax.fori_loop(0, n_trips, body, ())
    for i in range(3 * n_trips, n_chunks):
        step(i, i % 3)
    drain_write(n_chunks - 2, rows_bufs[(n_chunks - 2) % 3])
    drain_write(n_chunks - 1, rows_bufs[(n_chunks - 1) % 3])


@functools.partial(jax.jit, static_argnames=("batch",))
def _embedding_gather(flat_ids, table, batch):
    mesh = plsc.VectorSubcoreMesh(core_axis_name="c", subcore_axis_name="s")
    kern = functools.partial(
        pl.kernel,
        out_type=jax.ShapeDtypeStruct((batch, SEQ, PADDED_DIM), jnp.float32),
        mesh=mesh,
        compiler_params=pltpu.CompilerParams(use_tc_tiling_on_sc=False),
        scratch_types=[
            pltpu.VMEM((_CHUNK,), jnp.int32),
            pltpu.VMEM((_CHUNK,), jnp.int32),
            pltpu.VMEM((_CHUNK,), jnp.int32),
            pltpu.VMEM((_NB, SEQ, EMBED_DIM), jnp.float32),
            pltpu.VMEM((_NB, SEQ, EMBED_DIM), jnp.float32),
            pltpu.VMEM((_NB, SEQ, EMBED_DIM), jnp.float32),
            pltpu.SemaphoreType.DMA,
            pltpu.SemaphoreType.DMA,
            pltpu.SemaphoreType.DMA,
        ],
    )(functools.partial(_gather_kernel, batch))
    return kern(flat_ids, table)


def kernel(token_ids, emb0, emb1, emb2):
    B, T = token_ids.shape
    flat_ids = token_ids.reshape(B * T).astype(jnp.int32)
    out_padded = _embedding_gather(flat_ids, emb0, B)
    return out_padded[:, :, :EMBED_DIM]


# speedup vs baseline: 7.5574x; 1.0017x over previous
import functools

import jax
import jax.numpy as jnp
from jax import lax
from jax.experimental import pallas as pl
from jax.experimental.pallas import tpu as pltpu
from jax.experimental.pallas import tpu_sc as plsc

BASE_VOCAB = 100000
EMBED_DIM = 64
PADDED_DIM = 128
SEQ = 200

_info = plsc.get_sparse_core_info()
_NC, _NS, _L = _info.num_cores, _info.num_subcores, _info.num_lanes
_NW = _NC * _NS

_NB = 2
_CHUNK = _NB * SEQ
_SPLITS = ((0, 200),)


def _gather_kernel(batch, idx_hbm, table_hbm, out_hbm, idx0, idx1, idx2,
                   rows0, rows1, rows2, isem, gsem, wsem):
    b_per_w = batch // _NW
    n_chunks = b_per_w // _NB
    wid = lax.axis_index("s") * _NC + lax.axis_index("c")
    b_base = wid * b_per_w

    vmax = jnp.full((_L,), BASE_VOCAB - 1, dtype=jnp.int32)
    vmin = jnp.zeros((_L,), dtype=jnp.int32)

    idx_bufs = (idx0, idx1, idx2)
    rows_bufs = (rows0, rows1, rows2)

    def idx_load(i, idx_v):
        return pltpu.async_copy(
            idx_hbm.at[pl.ds((b_base + i * _NB) * SEQ, _CHUNK)], idx_v, isem)

    def drain_idx(i, idx_v):
        pltpu.make_async_copy(
            idx_hbm.at[pl.ds((b_base + i * _NB) * SEQ, _CHUNK)], idx_v,
            isem).wait()

    def clamp_fire(i, idx_v, rows_v):
        for c in range(_CHUNK // _L):
            sl = pl.ds(c * _L, _L)
            idx_v[sl] = jnp.minimum(jnp.maximum(idx_v[sl], vmin), vmax)
        for b in range(_NB):
            for (t0, n) in _SPLITS:
                pltpu.async_copy(
                    table_hbm.at[idx_v.at[pl.ds(b * SEQ + t0, n)]],
                    rows_v.at[b].at[pl.ds(t0, n)],
                    gsem)

    def out_slice(i):
        return out_hbm.at[pl.ds(b_base + i * _NB, _NB)].at[:, :, pl.ds(0, EMBED_DIM)]

    def wait_gathers(i, rows_v):
        pltpu.make_async_copy(out_slice(i), rows_v, gsem).wait()

    def write_out(i, rows_v):
        pltpu.async_copy(rows_v, out_slice(i), wsem)

    def drain_write(i, rows_v):
        pltpu.make_async_copy(rows_v, out_slice(i), wsem).wait()

    idx_load(0, idx0).wait()
    clamp_fire(0, idx0, rows0)
    idx_load(1, idx1)

    def step(i, q):
        @pl.when(i >= 2)
        def _():
            drain_write(i - 2, rows_bufs[(q + 1) % 3])

        @pl.when(i < n_chunks - 1)
        def _():
            drain_idx(i + 1, idx_bufs[(q + 1) % 3])
            clamp_fire(i + 1, idx_bufs[(q + 1) % 3], rows_bufs[(q + 1) % 3])

        wait_gathers(i, rows_bufs[q])

        @pl.when(i < n_chunks - 2)
        def _():
            idx_load(i + 2, idx_bufs[(q + 2) % 3])

        write_out(i, rows_bufs[q])

    n_trips = (n_chunks - 1) // 3

    def body(p, _):
        for q in range(3):
            step(3 * p + q, q)
        return ()

    lax.fori_loop(0, n_trips, body, ())
    for i in range(3 * n_trips, n_chunks):
        step(i, i % 3)
    drain_write(n_chunks - 2, rows_bufs[(n_chunks - 2) % 3])
    drain_write(n_chunks - 1, rows_bufs[(n_chunks - 1) % 3])


@functools.partial(jax.jit, static_argnames=("batch",))
def _embedding_gather(flat_ids, table, batch):
    mesh = plsc.VectorSubcoreMesh(core_axis_name="c", subcore_axis_name="s")
    kern = functools.partial(
        pl.kernel,
        out_type=jax.ShapeDtypeStruct((batch, SEQ, PADDED_DIM), jnp.float32),
        mesh=mesh,
        compiler_params=pltpu.CompilerParams(use_tc_tiling_on_sc=False),
        scratch_types=[
            pltpu.VMEM((_CHUNK,), jnp.int32),
            pltpu.VMEM((_CHUNK,), jnp.int32),
            pltpu.VMEM((_CHUNK,), jnp.int32),
            pltpu.VMEM((_NB, SEQ, EMBED_DIM), jnp.float32),
            pltpu.VMEM((_NB, SEQ, EMBED_DIM), jnp.float32),
            pltpu.VMEM((_NB, SEQ, EMBED_DIM), jnp.float32),
            pltpu.SemaphoreType.DMA,
            pltpu.SemaphoreType.DMA,
            pltpu.SemaphoreType.DMA,
        ],
    )(functools.partial(_gather_kernel, batch))
    return kern(flat_ids, table)


def kernel(token_ids, emb0, emb1, emb2):
    B, T = token_ids.shape
    flat_ids = token_ids.reshape(B * T).astype(jnp.int32)
    out_padded = _embedding_gather(flat_ids, emb0, B)
    return out_padded[:, :, :EMBED_DIM]
